# baseline (speedup 1.0000x reference)
import jax
import jax.numpy as jnp
from jax.experimental import pallas as pl
from jax.experimental.pallas import tpu as pltpu

NUM_EXPERTS = 64
TOP_K = 8
BLOCK_T = 256


def _router_block(x_ref, w_ref, wout_ref, iout_ref, lout_ref):
    x = x_ref[...]
    w = w_ref[...]
    logits = jax.lax.dot_general(
        x, w, (((1,), (1,)), ((), ())),
        preferred_element_type=jnp.float32,
    )
    lout_ref[...] = logits

    m = jnp.max(logits, axis=-1, keepdims=True)
    e = jnp.exp(logits - m)
    s = e / jnp.sum(e, axis=-1, keepdims=True)

    bt = s.shape[0]
    iota = jax.lax.broadcasted_iota(jnp.int32, (bt, NUM_EXPERTS), 1)
    total = jnp.zeros((bt, 1), jnp.float32)
    vals = []
    idxs = []
    cur = s
    for _ in range(TOP_K):
        mk = jnp.max(cur, axis=-1, keepdims=True)
        ik = jnp.min(jnp.where(cur == mk, iota, NUM_EXPERTS),
                     axis=-1, keepdims=True)
        vals.append(mk)
        idxs.append(ik)
        total = total + mk
        cur = jnp.where(iota == ik, -jnp.inf, cur)
    wout_ref[...] = jnp.concatenate(vals, axis=1) / total
    iout_ref[...] = jnp.concatenate(idxs, axis=1)


def kernel(x, W):
    n_tokens, d_model = x.shape
    grid = (n_tokens // BLOCK_T,)
    out_shapes = (
        jax.ShapeDtypeStruct((n_tokens, TOP_K), jnp.float32),
        jax.ShapeDtypeStruct((n_tokens, TOP_K), jnp.int32),
        jax.ShapeDtypeStruct((n_tokens, NUM_EXPERTS), jnp.float32),
    )
    return pl.pallas_call(
        _router_block,
        grid=grid,
        in_specs=[
            pl.BlockSpec((BLOCK_T, d_model), lambda i: (i, 0)),
            pl.BlockSpec((NUM_EXPERTS, d_model), lambda i: (0, 0)),
        ],
        out_specs=(
            pl.BlockSpec((BLOCK_T, TOP_K), lambda i: (i, 0)),
            pl.BlockSpec((BLOCK_T, TOP_K), lambda i: (i, 0)),
            pl.BlockSpec((BLOCK_T, NUM_EXPERTS), lambda i: (i, 0)),
        ),
        out_shape=out_shapes,
        compiler_params=pltpu.CompilerParams(
            dimension_semantics=("arbitrary",),
        ),
    )(x, W)


# speedup vs baseline: 1.5030x; 1.5030x over previous
import jax
import jax.numpy as jnp
from jax.experimental import pallas as pl
from jax.experimental.pallas import tpu as pltpu

NUM_EXPERTS = 64
TOP_K = 8
BLOCK_T = 256
_INT_MIN = -(2**31)


def _router_block(x_ref, w_ref, wout_ref, iout_ref, lout_ref):
    x = x_ref[...]
    w = w_ref[...]
    logits_t = jax.lax.dot_general(
        w, x, (((1,), (1,)), ((), ())),
        preferred_element_type=jnp.float32,
    )
    lout_ref[...] = logits_t.T

    bt = logits_t.shape[1]
    bits = jax.lax.bitcast_convert_type(logits_t, jnp.int32)
    okey = jnp.where(bits >= 0, bits, jnp.int32(_INT_MIN) - bits)
    iota = jax.lax.broadcasted_iota(jnp.int32, (NUM_EXPERTS, bt), 0)
    key = (okey & jnp.int32(~63)) | (jnp.int32(63) - iota)

    tops = []
    for _ in range(TOP_K):
        mk = jnp.max(key, axis=0, keepdims=True)
        tops.append(mk)
        key = jnp.where(key == mk, jnp.int32(_INT_MIN), key)

    top = jnp.concatenate(tops, axis=0)
    idx = jnp.int32(63) - (top & jnp.int32(63))
    vkey = top & jnp.int32(~63)
    vbits = jnp.where(vkey >= 0, vkey, jnp.int32(_INT_MIN) - vkey)
    lsel = jax.lax.bitcast_convert_type(vbits, jnp.float32)
    e = jnp.exp(lsel - lsel[:1, :])
    wsel = e / jnp.sum(e, axis=0, keepdims=True)
    wout_ref[...] = wsel.T
    iout_ref[...] = idx.T


def kernel(x, W):
    n_tokens, d_model = x.shape
    grid = (n_tokens // BLOCK_T,)
    out_shapes = (
        jax.ShapeDtypeStruct((n_tokens, TOP_K), jnp.float32),
        jax.ShapeDtypeStruct((n_tokens, TOP_K), jnp.int32),
        jax.ShapeDtypeStruct((n_tokens, NUM_EXPERTS), jnp.float32),
    )
    return pl.pallas_call(
        _router_block,
        grid=grid,
        in_specs=[
            pl.BlockSpec((BLOCK_T, d_model), lambda i: (i, 0)),
            pl.BlockSpec((NUM_EXPERTS, d_model), lambda i: (0, 0)),
        ],
        out_specs=(
            pl.BlockSpec((BLOCK_T, TOP_K), lambda i: (i, 0)),
            pl.BlockSpec((BLOCK_T, TOP_K), lambda i: (i, 0)),
            pl.BlockSpec((BLOCK_T, NUM_EXPERTS), lambda i: (i, 0)),
        ),
        out_shape=out_shapes,
        compiler_params=pltpu.CompilerParams(
            dimension_semantics=("arbitrary",),
        ),
    )(x, W)


# speedup vs baseline: 1.7813x; 1.1852x over previous
import jax
import jax.numpy as jnp
from jax.experimental import pallas as pl
from jax.experimental.pallas import tpu as pltpu

NUM_EXPERTS = 64
TOP_K = 8
BLOCK_T = 512
_INT_MIN = -(2**31)


def _router_block(x_ref, w_ref, wout_ref, iout_ref, lout_ref):
    x = x_ref[...]
    w = w_ref[...]
    logits_t = jax.lax.dot_general(
        w, x, (((1,), (1,)), ((), ())),
        preferred_element_type=jnp.float32,
    )
    lout_ref[...] = logits_t.T

    bt = logits_t.shape[1]
    bits = jax.lax.bitcast_convert_type(logits_t, jnp.int32)
    okey = jnp.where(bits >= 0, bits, jnp.int32(_INT_MIN) - bits)
    iota = jax.lax.broadcasted_iota(jnp.int32, (NUM_EXPERTS, bt), 0)
    key = (okey & jnp.int32(~63)) | (jnp.int32(63) - iota)

    tops = []
    for _ in range(TOP_K):
        mk = jnp.max(key, axis=0, keepdims=True)
        tops.append(mk)
        key = jnp.where(key == mk, jnp.int32(_INT_MIN), key)

    top = jnp.concatenate(tops, axis=0)
    idx = jnp.int32(63) - (top & jnp.int32(63))
    vkey = top & jnp.int32(~63)
    vbits = jnp.where(vkey >= 0, vkey, jnp.int32(_INT_MIN) - vkey)
    lsel = jax.lax.bitcast_convert_type(vbits, jnp.float32)
    e = jnp.exp(lsel - lsel[:1, :])
    wsel = e / jnp.sum(e, axis=0, keepdims=True)
    wout_ref[...] = wsel.T
    iout_ref[...] = idx.T


def kernel(x, W):
    n_tokens, d_model = x.shape
    grid = (n_tokens // BLOCK_T,)
    out_shapes = (
        jax.ShapeDtypeStruct((n_tokens, TOP_K), jnp.float32),
        jax.ShapeDtypeStruct((n_tokens, TOP_K), jnp.int32),
        jax.ShapeDtypeStruct((n_tokens, NUM_EXPERTS), jnp.float32),
    )
    return pl.pallas_call(
        _router_block,
        grid=grid,
        in_specs=[
            pl.BlockSpec((BLOCK_T, d_model), lambda i: (i, 0)),
            pl.BlockSpec((NUM_EXPERTS, d_model), lambda i: (0, 0)),
        ],
        out_specs=(
            pl.BlockSpec((BLOCK_T, TOP_K), lambda i: (i, 0)),
            pl.BlockSpec((BLOCK_T, TOP_K), lambda i: (i, 0)),
            pl.BlockSpec((BLOCK_T, NUM_EXPERTS), lambda i: (i, 0)),
        ),
        out_shape=out_shapes,
        compiler_params=pltpu.CompilerParams(
            dimension_semantics=("arbitrary",),
        ),
    )(x, W)


# speedup vs baseline: 1.8770x; 1.0537x over previous
import jax
import jax.numpy as jnp
from jax.experimental import pallas as pl
from jax.experimental.pallas import tpu as pltpu

NUM_EXPERTS = 64
TOP_K = 8
BLOCK_T = 1024
_INT_MIN = -(2**31)


def _router_block(x_ref, w_ref, wout_ref, iout_ref, lout_ref):
    x = x_ref[...]
    w = w_ref[...]
    logits_t = jax.lax.dot_general(
        w, x, (((1,), (1,)), ((), ())),
        preferred_element_type=jnp.float32,
    )
    lout_ref[...] = logits_t.T

    bt = logits_t.shape[1]
    bits = jax.lax.bitcast_convert_type(logits_t, jnp.int32)
    okey = jnp.where(bits >= 0, bits, jnp.int32(_INT_MIN) - bits)
    iota = jax.lax.broadcasted_iota(jnp.int32, (NUM_EXPERTS, bt), 0)
    key = (okey & jnp.int32(~63)) | (jnp.int32(63) - iota)

    tops = []
    for _ in range(TOP_K):
        mk = jnp.max(key, axis=0, keepdims=True)
        tops.append(mk)
        key = jnp.where(key == mk, jnp.int32(_INT_MIN), key)

    top = jnp.concatenate(tops, axis=0)
    idx = jnp.int32(63) - (top & jnp.int32(63))
    vkey = top & jnp.int32(~63)
    vbits = jnp.where(vkey >= 0, vkey, jnp.int32(_INT_MIN) - vkey)
    lsel = jax.lax.bitcast_convert_type(vbits, jnp.float32)
    e = jnp.exp(lsel - lsel[:1, :])
    wsel = e / jnp.sum(e, axis=0, keepdims=True)
    wout_ref[...] = wsel.T
    iout_ref[...] = idx.T


def kernel(x, W):
    n_tokens, d_model = x.shape
    grid = (n_tokens // BLOCK_T,)
    out_shapes = (
        jax.ShapeDtypeStruct((n_tokens, TOP_K), jnp.float32),
        jax.ShapeDtypeStruct((n_tokens, TOP_K), jnp.int32),
        jax.ShapeDtypeStruct((n_tokens, NUM_EXPERTS), jnp.float32),
    )
    return pl.pallas_call(
        _router_block,
        grid=grid,
        in_specs=[
            pl.BlockSpec((BLOCK_T, d_model), lambda i: (i, 0)),
            pl.BlockSpec((NUM_EXPERTS, d_model), lambda i: (0, 0)),
        ],
        out_specs=(
            pl.BlockSpec((BLOCK_T, TOP_K), lambda i: (i, 0)),
            pl.BlockSpec((BLOCK_T, TOP_K), lambda i: (i, 0)),
            pl.BlockSpec((BLOCK_T, NUM_EXPERTS), lambda i: (i, 0)),
        ),
        out_shape=out_shapes,
        compiler_params=pltpu.CompilerParams(
            dimension_semantics=("arbitrary",),
        ),
    )(x, W)


# speedup vs baseline: 1.8805x; 1.0019x over previous
import jax
import jax.numpy as jnp
from jax.experimental import pallas as pl
from jax.experimental.pallas import tpu as pltpu

NUM_EXPERTS = 64
TOP_K = 8
BLOCK_T = 1024
_INT_MIN = -(2**31)


def _router_block(xa_ref, xb_ref, xc_ref, xd_ref, w_ref,
                  wout_ref, iout_ref, lout_ref):
    w = w_ref[...]
    kq = xa_ref.shape[1]
    logits_t = jnp.zeros((NUM_EXPERTS, xa_ref.shape[0]), jnp.float32)
    for j, xr in enumerate((xa_ref, xb_ref, xc_ref, xd_ref)):
        logits_t = logits_t + jax.lax.dot_general(
            w[:, j * kq:(j + 1) * kq], xr[...],
            (((1,), (1,)), ((), ())),
            preferred_element_type=jnp.float32,
        )
    lout_ref[...] = logits_t.T

    bt = logits_t.shape[1]
    bits = jax.lax.bitcast_convert_type(logits_t, jnp.int32)
    okey = jnp.where(bits >= 0, bits, jnp.int32(_INT_MIN) - bits)
    iota = jax.lax.broadcasted_iota(jnp.int32, (NUM_EXPERTS, bt), 0)
    key = (okey & jnp.int32(~63)) | (jnp.int32(63) - iota)

    tops = []
    for _ in range(TOP_K):
        mk = jnp.max(key, axis=0, keepdims=True)
        tops.append(mk)
        key = jnp.where(key == mk, jnp.int32(_INT_MIN), key)

    top = jnp.concatenate(tops, axis=0)
    idx = jnp.int32(63) - (top & jnp.int32(63))
    vkey = top & jnp.int32(~63)
    vbits = jnp.where(vkey >= 0, vkey, jnp.int32(_INT_MIN) - vkey)
    lsel = jax.lax.bitcast_convert_type(vbits, jnp.float32)
    e = jnp.exp(lsel - lsel[:1, :])
    wsel = e / jnp.sum(e, axis=0, keepdims=True)
    wout_ref[...] = wsel.T
    iout_ref[...] = idx.T


def kernel(x, W):
    n_tokens, d_model = x.shape
    grid = (n_tokens // BLOCK_T,)
    out_shapes = (
        jax.ShapeDtypeStruct((n_tokens, TOP_K), jnp.float32),
        jax.ShapeDtypeStruct((n_tokens, TOP_K), jnp.int32),
        jax.ShapeDtypeStruct((n_tokens, NUM_EXPERTS), jnp.float32),
    )
    return pl.pallas_call(
        _router_block,
        grid=grid,
        in_specs=[
            pl.BlockSpec((BLOCK_T, d_model // 4), lambda i: (i, 0)),
            pl.BlockSpec((BLOCK_T, d_model // 4), lambda i: (i, 1)),
            pl.BlockSpec((BLOCK_T, d_model // 4), lambda i: (i, 2)),
            pl.BlockSpec((BLOCK_T, d_model // 4), lambda i: (i, 3)),
            pl.BlockSpec((NUM_EXPERTS, d_model), lambda i: (0, 0)),
        ],
        out_specs=(
            pl.BlockSpec((BLOCK_T, TOP_K), lambda i: (i, 0)),
            pl.BlockSpec((BLOCK_T, TOP_K), lambda i: (i, 0)),
            pl.BlockSpec((BLOCK_T, NUM_EXPERTS), lambda i: (i, 0)),
        ),
        out_shape=out_shapes,
        compiler_params=pltpu.CompilerParams(
            dimension_semantics=("arbitrary",),
        ),
    )(x, x, x, x, W)
